# baseline (speedup 1.0000x reference)
import functools
import math

import jax
import jax.numpy as jnp
from jax import lax
from jax.experimental import pallas as pl
from jax.experimental.pallas import tpu as pltpu
from jax.experimental.pallas import tpu_sc as plsc

N_EXPERTS = 64
CAPACITY_FACTOR = 1.25
AUX_COEF = 0.01

_CH = 1024
_SUB = 256
_NW = 32
_NSEG = 2


def _tc_body(S, x_ref, wt_ref, ci_ref, pi_ref, zi_ref,
             idx_ref, gate_ref, hist_ref, cnt_out, p_out, z_out, aux_ref,
             cnt_acc, p_acc, z_acc):
    i = pl.program_id(0)
    E = wt_ref.shape[1]

    @pl.when(i == 0)
    def _init():
        cnt_acc[...] = ci_ref[...]
        p_acc[...] = pi_ref[...]
        z_acc[0] = zi_ref[0]

    l = jnp.dot(x_ref[...], wt_ref[...], preferred_element_type=jnp.float32)
    m = jnp.max(l, axis=1, keepdims=True)
    ex = jnp.exp(l - m)
    s = jnp.sum(ex, axis=1, keepdims=True)
    idx = jnp.argmax(l, axis=1).astype(jnp.int32)
    r = 1.0 / s
    idx_ref[0, 0, :] = idx
    gate_ref[0, 0, :] = r[:, 0]

    p_acc[...] += jnp.sum(ex * r, axis=0, keepdims=True)
    oh = (lax.broadcasted_iota(jnp.int32, l.shape, 1)
          == idx[:, None]).astype(jnp.float32)
    for j in range(_CH // _SUB):
        h = jnp.sum(oh[j * _SUB:(j + 1) * _SUB, :], axis=0, keepdims=True)
        cnt_acc[...] += h
        hist_ref[0, j, :] = h[0].astype(jnp.int32)

    logz = m[:, 0] + jnp.log(s[:, 0])
    z_acc[0] += jnp.sum(logz * logz)

    @pl.when(i == pl.num_programs(0) - 1)
    def _fin():
        cnt_out[...] = cnt_acc[...]
        p_out[...] = p_acc[...]
        z_out[0] = z_acc[0]
        zl = AUX_COEF * z_acc[0] / S
        lb = (AUX_COEF * E * jnp.sum(cnt_acc[...] * p_acc[...])
              / (jnp.float32(S) * jnp.float32(S)))
        aux_ref[0] = zl + lb


def _tc_call(x_seg, wt, cnt0, p0, z0, S):
    sseg, D = x_seg.shape
    E = wt.shape[1]
    nch = sseg // _CH
    return pl.pallas_call(
        functools.partial(_tc_body, S),
        grid=(nch,),
        in_specs=[
            pl.BlockSpec((_CH, D), lambda i: (i, 0)),
            pl.BlockSpec((D, E), lambda i: (0, 0)),
            pl.BlockSpec((1, E), lambda i: (0, 0)),
            pl.BlockSpec((1, E), lambda i: (0, 0)),
            pl.BlockSpec(memory_space=pltpu.MemorySpace.SMEM),
        ],
        out_specs=[
            pl.BlockSpec((1, 1, _CH), lambda i: (i, 0, 0)),
            pl.BlockSpec((1, 1, _CH), lambda i: (i, 0, 0)),
            pl.BlockSpec((1, _CH // _SUB, E), lambda i: (i, 0, 0)),
            pl.BlockSpec((1, E), lambda i: (0, 0)),
            pl.BlockSpec((1, E), lambda i: (0, 0)),
            pl.BlockSpec(memory_space=pltpu.MemorySpace.SMEM),
            pl.BlockSpec(memory_space=pltpu.MemorySpace.SMEM),
        ],
        out_shape=[
            jax.ShapeDtypeStruct((nch, 1, _CH), jnp.int32),
            jax.ShapeDtypeStruct((nch, 1, _CH), jnp.float32),
            jax.ShapeDtypeStruct((nch, _CH // _SUB, E), jnp.int32),
            jax.ShapeDtypeStruct((1, E), jnp.float32),
            jax.ShapeDtypeStruct((1, E), jnp.float32),
            jax.ShapeDtypeStruct((1,), jnp.float32),
            jax.ShapeDtypeStruct((1,), jnp.float32),
        ],
        scratch_shapes=[
            pltpu.VMEM((1, E), jnp.float32),
            pltpu.VMEM((1, E), jnp.float32),
            pltpu.SMEM((1,), jnp.float32),
        ],
        compiler_params=pltpu.CompilerParams(
            dimension_semantics=("arbitrary",)),
    )(x_seg, wt, cnt0, p0, z0)


def _make_sc_scan(sseg, nhist, row0, capacity):
    E = N_EXPERTS
    sub = sseg // _NW
    mesh = plsc.VectorSubcoreMesh(core_axis_name="c", subcore_axis_name="s")

    @functools.partial(
        pl.kernel,
        mesh=mesh,
        compiler_params=pltpu.CompilerParams(needs_layout_passes=False),
        out_type=jax.ShapeDtypeStruct((sseg,), jnp.float32),
        scratch_types=[
            pltpu.VMEM((nhist * E,), jnp.int32),
            pltpu.VMEM((sub,), jnp.int32),
            pltpu.VMEM((sub,), jnp.float32),
            pltpu.VMEM((E,), jnp.int32),
            pltpu.VMEM((sub,), jnp.float32),
        ],
    )
    def scan(e_hbm, g_hbm, h_hbm, out_hbm, all_v, e_v, g_v, cnt_v, out_v):
        wid = lax.axis_index("s") * 2 + lax.axis_index("c")
        base = wid * sub
        pltpu.sync_copy(h_hbm, all_v)
        pltpu.sync_copy(e_hbm.at[pl.ds(base, sub)], e_v)
        pltpu.sync_copy(g_hbm.at[pl.ds(base, sub)], g_v)

        def bb(t, acc):
            return tuple(a + all_v[pl.ds(t * E + 16 * j, 16)]
                         for j, a in enumerate(acc))
        z16 = jnp.zeros((16,), jnp.int32)
        acc = lax.fori_loop(0, row0 + wid, bb, (z16,) * (E // 16))
        for j in range(E // 16):
            cnt_v[pl.ds(16 * j, 16)] = acc[j]

        lane = lax.broadcasted_iota(jnp.int32, (16,), 0)
        rots = [jnp.mod(lane - k, 16) for k in range(1, 16)]

        def sb(i, carry):
            ev = e_v[pl.ds(i * 16, 16)]
            gv = g_v[pl.ds(i * 16, 16)]
            cb = plsc.load_gather(cnt_v, [ev])
            rank = jnp.zeros((16,), jnp.int32)
            tot = jnp.zeros((16,), jnp.int32)
            for k in range(1, 16):
                eq = (ev == jnp.take_along_axis(
                    ev, rots[k - 1], axis=0, mode="promise_in_bounds")
                      ).astype(jnp.int32)
                tot = tot + eq
                rank = rank + jnp.where(lane >= k, eq, 0)
            pos = cb + rank + 1
            plsc.store_scatter(cnt_v, [ev], cb + tot + 1)
            out_v[pl.ds(i * 16, 16)] = jnp.where(
                pos <= capacity, gv, jnp.float32(0.0))
            return carry
        lax.fori_loop(0, sub // 16, sb, 0)
        pltpu.sync_copy(out_v, out_hbm.at[pl.ds(base, sub)])

    return scan


def kernel(x, W):
    B, T, D = x.shape
    S = B * T
    E = W.shape[0]
    capacity = math.ceil(S / E * CAPACITY_FACTOR)
    sseg = S // _NSEG

    x2 = x.reshape(S, D)
    wt = W.T

    cnt = jnp.zeros((1, E), jnp.float32)
    p = jnp.zeros((1, E), jnp.float32)
    z = jnp.zeros((1,), jnp.float32)
    idxs, kepts, hists = [], [], []
    aux = None
    for h in range(_NSEG):
        x_seg = x2[h * sseg:(h + 1) * sseg]
        idx2, gate2, hist, cnt, p, z, aux = _tc_call(x_seg, wt, cnt, p, z, S)
        hists.append(hist.reshape(-1))
        e_flat = idx2.reshape(sseg)
        g_flat = gate2.reshape(sseg)
        h_flat = jnp.concatenate(hists) if h else hists[0]
        kept = _make_sc_scan(sseg, (h + 1) * _NW, h * _NW, capacity)(
            e_flat, g_flat, h_flat)
        idxs.append(e_flat)
        kepts.append(kept)
    return (jnp.concatenate(idxs), jnp.concatenate(kepts), aux[0])


# speedup vs baseline: 1.7956x; 1.7956x over previous
import functools
import math

import jax
import jax.numpy as jnp
from jax import lax
from jax.experimental import pallas as pl
from jax.experimental.pallas import tpu as pltpu
from jax.experimental.pallas import tpu_sc as plsc

N_EXPERTS = 64
CAPACITY_FACTOR = 1.25
AUX_COEF = 0.01

_CH = 1024
_SUB = 256
_NW = 32
_NSEG = 2


def _tc_body(S, x_ref, wt_ref, ci_ref, pi_ref, zi_ref,
             idx_ref, gate_ref, hist_ref, cnt_out, p_out, z_out, aux_ref,
             cnt_acc, p_acc, z_acc):
    i = pl.program_id(0)
    E = wt_ref.shape[1]

    @pl.when(i == 0)
    def _init():
        cnt_acc[...] = ci_ref[...]
        p_acc[...] = pi_ref[...]
        z_acc[0] = zi_ref[0]

    l = jnp.dot(x_ref[...], wt_ref[...], preferred_element_type=jnp.float32)
    m = jnp.max(l, axis=1, keepdims=True)
    ex = jnp.exp(l - m)
    s = jnp.sum(ex, axis=1, keepdims=True)
    idx = jnp.argmax(l, axis=1).astype(jnp.int32)
    r = 1.0 / s
    idx_ref[0, 0, :] = idx
    gate_ref[0, 0, :] = r[:, 0]

    p_acc[...] += jnp.sum(ex * r, axis=0, keepdims=True)
    oh = (lax.broadcasted_iota(jnp.int32, l.shape, 1)
          == idx[:, None]).astype(jnp.float32)
    for j in range(_CH // _SUB):
        h = jnp.sum(oh[j * _SUB:(j + 1) * _SUB, :], axis=0, keepdims=True)
        cnt_acc[...] += h
        hist_ref[0, j, :] = h[0].astype(jnp.int32)

    logz = m[:, 0] + jnp.log(s[:, 0])
    z_acc[0] += jnp.sum(logz * logz)

    @pl.when(i == pl.num_programs(0) - 1)
    def _fin():
        cnt_out[...] = cnt_acc[...]
        p_out[...] = p_acc[...]
        z_out[0] = z_acc[0]
        zl = AUX_COEF * z_acc[0] / S
        lb = (AUX_COEF * E * jnp.sum(cnt_acc[...] * p_acc[...])
              / (jnp.float32(S) * jnp.float32(S)))
        aux_ref[0] = zl + lb


def _tc_call(x2, wt, cnt0, p0, z0, S, h, sseg):
    D = x2.shape[1]
    E = wt.shape[1]
    nch = sseg // _CH
    off = h * nch
    return pl.pallas_call(
        functools.partial(_tc_body, S),
        grid=(nch,),
        in_specs=[
            pl.BlockSpec((_CH, D), lambda i: (i + off, 0)),
            pl.BlockSpec((D, E), lambda i: (0, 0)),
            pl.BlockSpec((1, E), lambda i: (0, 0)),
            pl.BlockSpec((1, E), lambda i: (0, 0)),
            pl.BlockSpec(memory_space=pltpu.MemorySpace.SMEM),
        ],
        out_specs=[
            pl.BlockSpec((1, 1, _CH), lambda i: (i, 0, 0)),
            pl.BlockSpec((1, 1, _CH), lambda i: (i, 0, 0)),
            pl.BlockSpec((1, _CH // _SUB, E), lambda i: (i, 0, 0)),
            pl.BlockSpec((1, E), lambda i: (0, 0)),
            pl.BlockSpec((1, E), lambda i: (0, 0)),
            pl.BlockSpec(memory_space=pltpu.MemorySpace.SMEM),
            pl.BlockSpec(memory_space=pltpu.MemorySpace.SMEM),
        ],
        out_shape=[
            jax.ShapeDtypeStruct((nch, 1, _CH), jnp.int32),
            jax.ShapeDtypeStruct((nch, 1, _CH), jnp.float32),
            jax.ShapeDtypeStruct((nch, _CH // _SUB, E), jnp.int32),
            jax.ShapeDtypeStruct((1, E), jnp.float32),
            jax.ShapeDtypeStruct((1, E), jnp.float32),
            jax.ShapeDtypeStruct((1,), jnp.float32),
            jax.ShapeDtypeStruct((1,), jnp.float32),
        ],
        scratch_shapes=[
            pltpu.VMEM((1, E), jnp.float32),
            pltpu.VMEM((1, E), jnp.float32),
            pltpu.SMEM((1,), jnp.float32),
        ],
        compiler_params=pltpu.CompilerParams(
            dimension_semantics=("arbitrary",)),
    )(x2, wt, cnt0, p0, z0)


def _make_sc_scan(sseg, nhist, row0, capacity):
    E = N_EXPERTS
    sub = sseg // _NW
    mesh = plsc.VectorSubcoreMesh(core_axis_name="c", subcore_axis_name="s")

    @functools.partial(
        pl.kernel,
        mesh=mesh,
        compiler_params=pltpu.CompilerParams(needs_layout_passes=False),
        out_type=jax.ShapeDtypeStruct((sseg,), jnp.float32),
        scratch_types=[
            pltpu.VMEM((nhist * E,), jnp.int32),
            pltpu.VMEM((sub,), jnp.int32),
            pltpu.VMEM((sub,), jnp.float32),
            pltpu.VMEM((E,), jnp.int32),
            pltpu.VMEM((sub,), jnp.float32),
        ],
    )
    def scan(e_hbm, g_hbm, h_hbm, out_hbm, all_v, e_v, g_v, cnt_v, out_v):
        wid = lax.axis_index("s") * 2 + lax.axis_index("c")
        base = wid * sub
        pltpu.sync_copy(h_hbm, all_v)
        pltpu.sync_copy(e_hbm.at[pl.ds(base, sub)], e_v)
        pltpu.sync_copy(g_hbm.at[pl.ds(base, sub)], g_v)

        def bb(t, acc):
            return tuple(a + all_v[pl.ds(t * E + 16 * j, 16)]
                         for j, a in enumerate(acc))
        z16 = jnp.zeros((16,), jnp.int32)
        acc = lax.fori_loop(0, row0 + wid, bb, (z16,) * (E // 16))
        for j in range(E // 16):
            cnt_v[pl.ds(16 * j, 16)] = acc[j]

        lane = lax.broadcasted_iota(jnp.int32, (16,), 0)
        rots = [jnp.mod(lane - k, 16) for k in range(1, 16)]

        def sb(i, carry):
            ev = e_v[pl.ds(i * 16, 16)]
            gv = g_v[pl.ds(i * 16, 16)]
            cb = plsc.load_gather(cnt_v, [ev])
            rank = jnp.zeros((16,), jnp.int32)
            tot = jnp.zeros((16,), jnp.int32)
            for k in range(1, 16):
                eq = (ev == jnp.take_along_axis(
                    ev, rots[k - 1], axis=0, mode="promise_in_bounds")
                      ).astype(jnp.int32)
                tot = tot + eq
                rank = rank + jnp.where(lane >= k, eq, 0)
            pos = cb + rank + 1
            plsc.store_scatter(cnt_v, [ev], cb + tot + 1)
            out_v[pl.ds(i * 16, 16)] = jnp.where(
                pos <= capacity, gv, jnp.float32(0.0))
            return carry
        lax.fori_loop(0, sub // 16, sb, 0)
        pltpu.sync_copy(out_v, out_hbm.at[pl.ds(base, sub)])

    return scan


def kernel(x, W):
    B, T, D = x.shape
    S = B * T
    E = W.shape[0]
    capacity = math.ceil(S / E * CAPACITY_FACTOR)
    sseg = S // _NSEG

    x2 = x.reshape(S, D)
    wt = W.T

    cnt = jnp.zeros((1, E), jnp.float32)
    p = jnp.zeros((1, E), jnp.float32)
    z = jnp.zeros((1,), jnp.float32)
    idxs, kepts, hists = [], [], []
    aux = None
    for h in range(_NSEG):
        idx2, gate2, hist, cnt, p, z, aux = _tc_call(
            x2, wt, cnt, p, z, S, h, sseg)
        hists.append(hist.reshape(-1))
        e_flat = idx2.reshape(sseg)
        g_flat = gate2.reshape(sseg)
        h_flat = jnp.concatenate(hists) if h else hists[0]
        kept = _make_sc_scan(sseg, (h + 1) * _NW, h * _NW, capacity)(
            e_flat, g_flat, h_flat)
        idxs.append(e_flat)
        kepts.append(kept)
    return (jnp.concatenate(idxs), jnp.concatenate(kepts), aux[0])


# speedup vs baseline: 2.1182x; 1.1797x over previous
import functools
import math

import jax
import jax.numpy as jnp
from jax import lax
from jax.experimental import pallas as pl
from jax.experimental.pallas import tpu as pltpu
from jax.experimental.pallas import tpu_sc as plsc

N_EXPERTS = 64
CAPACITY_FACTOR = 1.25
AUX_COEF = 0.01

_CH = 1024
_SUB = 512
_NW = 32
_NSEG = 1


def _tc_body(S, x_ref, wt_ref, ci_ref, pi_ref, zi_ref,
             idx_ref, gate_ref, hist_ref, cnt_out, p_out, z_out, aux_ref,
             cnt_acc, p_acc, z_acc):
    i = pl.program_id(0)
    E = wt_ref.shape[1]

    @pl.when(i == 0)
    def _init():
        cnt_acc[...] = ci_ref[...]
        p_acc[...] = pi_ref[...]
        z_acc[0] = zi_ref[0]

    l = jnp.dot(x_ref[...], wt_ref[...], preferred_element_type=jnp.float32)
    m = jnp.max(l, axis=1, keepdims=True)
    ex = jnp.exp(l - m)
    s = jnp.sum(ex, axis=1, keepdims=True)
    idx = jnp.argmax(l, axis=1).astype(jnp.int32)
    r = 1.0 / s
    idx_ref[0, 0, :] = idx
    gate_ref[0, 0, :] = r[:, 0]

    p_acc[...] += jnp.sum(ex * r, axis=0, keepdims=True)
    oh = (lax.broadcasted_iota(jnp.int32, l.shape, 1)
          == idx[:, None]).astype(jnp.float32)
    for j in range(_CH // _SUB):
        h = jnp.sum(oh[j * _SUB:(j + 1) * _SUB, :], axis=0, keepdims=True)
        cnt_acc[...] += h
        hist_ref[0, j, :] = h[0].astype(jnp.int32)

    logz = m[:, 0] + jnp.log(s[:, 0])
    z_acc[0] += jnp.sum(logz * logz)

    @pl.when(i == pl.num_programs(0) - 1)
    def _fin():
        cnt_out[...] = cnt_acc[...]
        p_out[...] = p_acc[...]
        z_out[0] = z_acc[0]
        zl = AUX_COEF * z_acc[0] / S
        lb = (AUX_COEF * E * jnp.sum(cnt_acc[...] * p_acc[...])
              / (jnp.float32(S) * jnp.float32(S)))
        aux_ref[0] = zl + lb


def _tc_call(x2, wt, cnt0, p0, z0, S, h, sseg):
    D = x2.shape[1]
    E = wt.shape[1]
    nch = sseg // _CH
    off = h * nch
    return pl.pallas_call(
        functools.partial(_tc_body, S),
        grid=(nch,),
        in_specs=[
            pl.BlockSpec((_CH, D), lambda i: (i + off, 0)),
            pl.BlockSpec((D, E), lambda i: (0, 0)),
            pl.BlockSpec((1, E), lambda i: (0, 0)),
            pl.BlockSpec((1, E), lambda i: (0, 0)),
            pl.BlockSpec(memory_space=pltpu.MemorySpace.SMEM),
        ],
        out_specs=[
            pl.BlockSpec((1, 1, _CH), lambda i: (i, 0, 0)),
            pl.BlockSpec((1, 1, _CH), lambda i: (i, 0, 0)),
            pl.BlockSpec((1, _CH // _SUB, E), lambda i: (i, 0, 0)),
            pl.BlockSpec((1, E), lambda i: (0, 0)),
            pl.BlockSpec((1, E), lambda i: (0, 0)),
            pl.BlockSpec(memory_space=pltpu.MemorySpace.SMEM),
            pl.BlockSpec(memory_space=pltpu.MemorySpace.SMEM),
        ],
        out_shape=[
            jax.ShapeDtypeStruct((nch, 1, _CH), jnp.int32),
            jax.ShapeDtypeStruct((nch, 1, _CH), jnp.float32),
            jax.ShapeDtypeStruct((nch, _CH // _SUB, E), jnp.int32),
            jax.ShapeDtypeStruct((1, E), jnp.float32),
            jax.ShapeDtypeStruct((1, E), jnp.float32),
            jax.ShapeDtypeStruct((1,), jnp.float32),
            jax.ShapeDtypeStruct((1,), jnp.float32),
        ],
        scratch_shapes=[
            pltpu.VMEM((1, E), jnp.float32),
            pltpu.VMEM((1, E), jnp.float32),
            pltpu.SMEM((1,), jnp.float32),
        ],
        compiler_params=pltpu.CompilerParams(
            dimension_semantics=("arbitrary",)),
    )(x2, wt, cnt0, p0, z0)


def _make_sc_scan(sseg, nhist, row0, capacity):
    E = N_EXPERTS
    sub = sseg // _NW
    rpt = sub // _SUB
    mesh = plsc.VectorSubcoreMesh(core_axis_name="c", subcore_axis_name="s")

    @functools.partial(
        pl.kernel,
        mesh=mesh,
        compiler_params=pltpu.CompilerParams(needs_layout_passes=False),
        out_type=jax.ShapeDtypeStruct((sseg,), jnp.float32),
        scratch_types=[
            pltpu.VMEM((nhist * E,), jnp.int32),
            pltpu.VMEM((sub,), jnp.int32),
            pltpu.VMEM((sub,), jnp.float32),
            pltpu.VMEM((E,), jnp.int32),
            pltpu.VMEM((sub,), jnp.float32),
        ],
    )
    def scan(e_hbm, g_hbm, h_hbm, out_hbm, all_v, e_v, g_v, cnt_v, out_v):
        wid = lax.axis_index("s") * 2 + lax.axis_index("c")
        base = wid * sub
        pltpu.sync_copy(h_hbm, all_v)
        pltpu.sync_copy(e_hbm.at[pl.ds(base, sub)], e_v)
        pltpu.sync_copy(g_hbm.at[pl.ds(base, sub)], g_v)

        def bb(t, acc):
            return tuple(a + all_v[pl.ds(t * E + 16 * j, 16)]
                         for j, a in enumerate(acc))
        z16 = jnp.zeros((16,), jnp.int32)
        acc = lax.fori_loop(0, row0 + wid * rpt, bb, (z16,) * (E // 16))
        for j in range(E // 16):
            cnt_v[pl.ds(16 * j, 16)] = acc[j]

        lane = lax.broadcasted_iota(jnp.int32, (16,), 0)
        rots = [jnp.mod(lane - k, 16) for k in range(1, 16)]

        def sb(i, carry):
            ev = e_v[pl.ds(i * 16, 16)]
            gv = g_v[pl.ds(i * 16, 16)]
            cb = plsc.load_gather(cnt_v, [ev])
            rank = jnp.zeros((16,), jnp.int32)
            tot = jnp.zeros((16,), jnp.int32)
            for k in range(1, 16):
                eq = (ev == jnp.take_along_axis(
                    ev, rots[k - 1], axis=0, mode="promise_in_bounds")
                      ).astype(jnp.int32)
                tot = tot + eq
                rank = rank + jnp.where(lane >= k, eq, 0)
            pos = cb + rank + 1
            plsc.store_scatter(cnt_v, [ev], cb + tot + 1)
            out_v[pl.ds(i * 16, 16)] = jnp.where(
                pos <= capacity, gv, jnp.float32(0.0))
            return carry
        lax.fori_loop(0, sub // 16, sb, 0)
        pltpu.sync_copy(out_v, out_hbm.at[pl.ds(base, sub)])

    return scan


def kernel(x, W):
    B, T, D = x.shape
    S = B * T
    E = W.shape[0]
    capacity = math.ceil(S / E * CAPACITY_FACTOR)
    sseg = S // _NSEG

    x2 = x.reshape(S, D)
    wt = W.T

    cnt = jnp.zeros((1, E), jnp.float32)
    p = jnp.zeros((1, E), jnp.float32)
    z = jnp.zeros((1,), jnp.float32)
    idxs, kepts, hists = [], [], []
    aux = None
    for h in range(_NSEG):
        idx2, gate2, hist, cnt, p, z, aux = _tc_call(
            x2, wt, cnt, p, z, S, h, sseg)
        hists.append(hist.reshape(-1))
        e_flat = idx2.reshape(sseg)
        g_flat = gate2.reshape(sseg)
        h_flat = jnp.concatenate(hists) if h else hists[0]
        nh = (h + 1) * sseg // _SUB
        kept = _make_sc_scan(sseg, nh, h * sseg // _SUB, capacity)(
            e_flat, g_flat, h_flat)
        idxs.append(e_flat)
        kepts.append(kept)
    return (jnp.concatenate(idxs), jnp.concatenate(kepts), aux[0])


# speedup vs baseline: 2.1260x; 1.0037x over previous
import functools
import math

import jax
import jax.numpy as jnp
from jax import lax
from jax.experimental import pallas as pl
from jax.experimental.pallas import tpu as pltpu
from jax.experimental.pallas import tpu_sc as plsc

N_EXPERTS = 64
CAPACITY_FACTOR = 1.25
AUX_COEF = 0.01

_CH = 1024
_SUB = 512
_NW = 32
_NSEG = 1


def _tc_body(S, x_ref, wt_ref, ci_ref, pi_ref, zi_ref,
             idx_ref, gate_ref, hist_ref, cnt_out, p_out, z_out, aux_ref,
             cnt_acc, p_acc, z_acc):
    i = pl.program_id(0)
    E = wt_ref.shape[1]

    @pl.when(i == 0)
    def _init():
        cnt_acc[...] = ci_ref[...]
        p_acc[...] = pi_ref[...]
        z_acc[0] = zi_ref[0]

    l = jnp.dot(x_ref[...], wt_ref[...], preferred_element_type=jnp.float32)
    m = jnp.max(l, axis=1, keepdims=True)
    ex = jnp.exp(l - m)
    s = jnp.sum(ex, axis=1, keepdims=True)
    idx = jnp.argmax(l, axis=1).astype(jnp.int32)
    r = 1.0 / s
    idx_ref[0, 0, :] = idx
    gate_ref[0, 0, :] = r[:, 0]

    p_acc[...] += jnp.sum(ex * r, axis=0, keepdims=True)
    oh = (lax.broadcasted_iota(jnp.int32, l.shape, 1)
          == idx[:, None]).astype(jnp.float32)
    for j in range(_CH // _SUB):
        h = jnp.sum(oh[j * _SUB:(j + 1) * _SUB, :], axis=0, keepdims=True)
        cnt_acc[...] += h
        hist_ref[0, j, :] = h[0].astype(jnp.int32)

    logz = m[:, 0] + jnp.log(s[:, 0])
    z_acc[0] += jnp.sum(logz * logz)

    @pl.when(i == pl.num_programs(0) - 1)
    def _fin():
        cnt_out[...] = cnt_acc[...]
        p_out[...] = p_acc[...]
        z_out[0] = z_acc[0]
        zl = AUX_COEF * z_acc[0] / S
        lb = (AUX_COEF * E * jnp.sum(cnt_acc[...] * p_acc[...])
              / (jnp.float32(S) * jnp.float32(S)))
        aux_ref[0] = zl + lb


def _tc_call(x2, wt, cnt0, p0, z0, S, h, sseg):
    D = x2.shape[1]
    E = wt.shape[1]
    nch = sseg // _CH
    off = h * nch
    return pl.pallas_call(
        functools.partial(_tc_body, S),
        grid=(nch,),
        in_specs=[
            pl.BlockSpec((_CH, D), lambda i: (i + off, 0)),
            pl.BlockSpec((D, E), lambda i: (0, 0)),
            pl.BlockSpec((1, E), lambda i: (0, 0)),
            pl.BlockSpec((1, E), lambda i: (0, 0)),
            pl.BlockSpec(memory_space=pltpu.MemorySpace.SMEM),
        ],
        out_specs=[
            pl.BlockSpec((1, 1, _CH), lambda i: (i, 0, 0)),
            pl.BlockSpec((1, 1, _CH), lambda i: (i, 0, 0)),
            pl.BlockSpec((1, _CH // _SUB, E), lambda i: (i, 0, 0)),
            pl.BlockSpec((1, E), lambda i: (0, 0)),
            pl.BlockSpec((1, E), lambda i: (0, 0)),
            pl.BlockSpec(memory_space=pltpu.MemorySpace.SMEM),
            pl.BlockSpec(memory_space=pltpu.MemorySpace.SMEM),
        ],
        out_shape=[
            jax.ShapeDtypeStruct((nch, 1, _CH), jnp.int32),
            jax.ShapeDtypeStruct((nch, 1, _CH), jnp.float32),
            jax.ShapeDtypeStruct((nch, _CH // _SUB, E), jnp.int32),
            jax.ShapeDtypeStruct((1, E), jnp.float32),
            jax.ShapeDtypeStruct((1, E), jnp.float32),
            jax.ShapeDtypeStruct((1,), jnp.float32),
            jax.ShapeDtypeStruct((1,), jnp.float32),
        ],
        scratch_shapes=[
            pltpu.VMEM((1, E), jnp.float32),
            pltpu.VMEM((1, E), jnp.float32),
            pltpu.SMEM((1,), jnp.float32),
        ],
        compiler_params=pltpu.CompilerParams(
            dimension_semantics=("arbitrary",)),
    )(x2, wt, cnt0, p0, z0)


def _make_sc_scan(sseg, nhist, row0, capacity):
    E = N_EXPERTS
    sub = sseg // _NW
    rpt = sub // _SUB
    mesh = plsc.VectorSubcoreMesh(core_axis_name="c", subcore_axis_name="s")

    @functools.partial(
        pl.kernel,
        mesh=mesh,
        compiler_params=pltpu.CompilerParams(needs_layout_passes=False),
        out_type=jax.ShapeDtypeStruct((sseg,), jnp.float32),
        scratch_types=[
            pltpu.VMEM((nhist * E,), jnp.int32),
            pltpu.VMEM((sub,), jnp.int32),
            pltpu.VMEM((sub,), jnp.float32),
            pltpu.VMEM((E,), jnp.int32),
            pltpu.VMEM((sub,), jnp.float32),
        ],
    )
    def scan(e_hbm, g_hbm, h_hbm, out_hbm, all_v, e_v, g_v, cnt_v, out_v):
        wid = lax.axis_index("s") * 2 + lax.axis_index("c")
        base = wid * sub
        pltpu.sync_copy(h_hbm, all_v)
        pltpu.sync_copy(e_hbm.at[pl.ds(base, sub)], e_v)
        pltpu.sync_copy(g_hbm.at[pl.ds(base, sub)], g_v)

        def bb(t, acc):
            return tuple(a + all_v[pl.ds(t * E + 16 * j, 16)]
                         for j, a in enumerate(acc))
        z16 = jnp.zeros((16,), jnp.int32)
        acc = lax.fori_loop(0, row0 + wid * rpt, bb, (z16,) * (E // 16))
        for j in range(E // 16):
            cnt_v[pl.ds(16 * j, 16)] = acc[j]

        def sb(i, carry):
            ev = e_v[pl.ds(i * 16, 16)]
            gv = g_v[pl.ds(i * 16, 16)]
            cb = plsc.load_gather(cnt_v, [ev])
            occ, last = plsc.scan_count(ev)
            pos = cb + occ
            plsc.store_scatter(cnt_v, [ev], pos, mask=last)
            out_v[pl.ds(i * 16, 16)] = jnp.where(
                pos <= capacity, gv, jnp.float32(0.0))
            return carry
        lax.fori_loop(0, sub // 16, sb, 0)
        pltpu.sync_copy(out_v, out_hbm.at[pl.ds(base, sub)])

    return scan


def kernel(x, W):
    B, T, D = x.shape
    S = B * T
    E = W.shape[0]
    capacity = math.ceil(S / E * CAPACITY_FACTOR)
    sseg = S // _NSEG

    x2 = x.reshape(S, D)
    wt = W.T

    cnt = jnp.zeros((1, E), jnp.float32)
    p = jnp.zeros((1, E), jnp.float32)
    z = jnp.zeros((1,), jnp.float32)
    idxs, kepts, hists = [], [], []
    aux = None
    for h in range(_NSEG):
        idx2, gate2, hist, cnt, p, z, aux = _tc_call(
            x2, wt, cnt, p, z, S, h, sseg)
        hists.append(hist.reshape(-1))
        e_flat = idx2.reshape(sseg)
        g_flat = gate2.reshape(sseg)
        h_flat = jnp.concatenate(hists) if h else hists[0]
        nh = (h + 1) * sseg // _SUB
        kept = _make_sc_scan(sseg, nh, h * sseg // _SUB, capacity)(
            e_flat, g_flat, h_flat)
        idxs.append(e_flat)
        kepts.append(kept)
    if _NSEG == 1:
        return (idxs[0], kepts[0], aux[0])
    return (jnp.concatenate(idxs), jnp.concatenate(kepts), aux[0])


# speedup vs baseline: 2.1482x; 1.0104x over previous
import functools
import math

import jax
import jax.numpy as jnp
from jax import lax
from jax.experimental import pallas as pl
from jax.experimental.pallas import tpu as pltpu
from jax.experimental.pallas import tpu_sc as plsc

N_EXPERTS = 64
CAPACITY_FACTOR = 1.25
AUX_COEF = 0.01

_CH = 1024
_SUB = 512
_NW = 32


def _tc_body(S, x_ref, wt_ref, idx_ref, gate_ref, hist_ref, aux_ref,
             cnt_acc, p_acc, z_acc):
    i = pl.program_id(0)
    E = wt_ref.shape[1]

    @pl.when(i == 0)
    def _init():
        cnt_acc[...] = jnp.zeros_like(cnt_acc)
        p_acc[...] = jnp.zeros_like(p_acc)
        z_acc[0] = jnp.float32(0.0)

    l = jnp.dot(x_ref[...], wt_ref[...], preferred_element_type=jnp.float32)
    m = jnp.max(l, axis=1, keepdims=True)
    ex = jnp.exp(l - m)
    s = jnp.sum(ex, axis=1, keepdims=True)
    idx = jnp.argmax(l, axis=1).astype(jnp.int32)
    r = 1.0 / s
    idx_ref[0, 0, :] = idx
    gate_ref[0, 0, :] = r[:, 0]

    p_acc[...] += jnp.sum(ex * r, axis=0, keepdims=True)
    oh = (lax.broadcasted_iota(jnp.int32, l.shape, 1)
          == idx[:, None]).astype(jnp.float32)
    for j in range(_CH // _SUB):
        h = jnp.sum(oh[j * _SUB:(j + 1) * _SUB, :], axis=0, keepdims=True)
        cnt_acc[...] += h
        hist_ref[0, j, :] = h[0].astype(jnp.int32)

    logz = m[:, 0] + jnp.log(s[:, 0])
    z_acc[0] += jnp.sum(logz * logz)

    @pl.when(i == pl.num_programs(0) - 1)
    def _fin():
        zl = AUX_COEF * z_acc[0] / S
        lb = (AUX_COEF * E * jnp.sum(cnt_acc[...] * p_acc[...])
              / (jnp.float32(S) * jnp.float32(S)))
        aux_ref[0] = zl + lb


def _tc_call(x2, wt):
    S, D = x2.shape
    E = wt.shape[1]
    nch = S // _CH
    return pl.pallas_call(
        functools.partial(_tc_body, S),
        grid=(nch,),
        in_specs=[
            pl.BlockSpec((_CH, D), lambda i: (i, 0)),
            pl.BlockSpec((D, E), lambda i: (0, 0)),
        ],
        out_specs=[
            pl.BlockSpec((1, 1, _CH), lambda i: (i, 0, 0)),
            pl.BlockSpec((1, 1, _CH), lambda i: (i, 0, 0)),
            pl.BlockSpec((1, _CH // _SUB, E), lambda i: (i, 0, 0)),
            pl.BlockSpec(memory_space=pltpu.MemorySpace.SMEM),
        ],
        out_shape=[
            jax.ShapeDtypeStruct((nch, 1, _CH), jnp.int32),
            jax.ShapeDtypeStruct((nch, 1, _CH), jnp.float32),
            jax.ShapeDtypeStruct((nch, _CH // _SUB, E), jnp.int32),
            jax.ShapeDtypeStruct((1,), jnp.float32),
        ],
        scratch_shapes=[
            pltpu.VMEM((1, E), jnp.float32),
            pltpu.VMEM((1, E), jnp.float32),
            pltpu.SMEM((1,), jnp.float32),
        ],
        compiler_params=pltpu.CompilerParams(
            dimension_semantics=("arbitrary",)),
    )(x2, wt)


def _make_sc_scan(S, capacity):
    E = N_EXPERTS
    sub = S // _NW
    rpt = sub // _SUB
    nhist = S // _SUB
    mesh = plsc.VectorSubcoreMesh(core_axis_name="c", subcore_axis_name="s")

    @functools.partial(
        pl.kernel,
        mesh=mesh,
        compiler_params=pltpu.CompilerParams(needs_layout_passes=False),
        out_type=jax.ShapeDtypeStruct((S,), jnp.float32),
        scratch_types=[
            pltpu.VMEM((nhist * E,), jnp.int32),
            pltpu.VMEM((sub,), jnp.int32),
            pltpu.VMEM((sub,), jnp.float32),
            pltpu.VMEM((E,), jnp.int32),
            pltpu.VMEM((sub,), jnp.float32),
        ],
    )
    def scan(e_hbm, g_hbm, h_hbm, out_hbm, all_v, e_v, g_v, cnt_v, out_v):
        wid = lax.axis_index("s") * 2 + lax.axis_index("c")
        base = wid * sub
        pltpu.sync_copy(h_hbm, all_v)
        pltpu.sync_copy(e_hbm.at[pl.ds(base, sub)], e_v)
        pltpu.sync_copy(g_hbm.at[pl.ds(base, sub)], g_v)

        def bb(t, acc):
            return tuple(a + all_v[pl.ds(t * E + 16 * j, 16)]
                         for j, a in enumerate(acc))
        z16 = jnp.zeros((16,), jnp.int32)
        acc = lax.fori_loop(0, wid * rpt, bb, (z16,) * (E // 16))
        for j in range(E // 16):
            cnt_v[pl.ds(16 * j, 16)] = acc[j]

        def sb(i, carry):
            ev = e_v[pl.ds(i * 16, 16)]
            gv = g_v[pl.ds(i * 16, 16)]
            cb = plsc.load_gather(cnt_v, [ev])
            occ, last = plsc.scan_count(ev)
            pos = cb + occ
            plsc.store_scatter(cnt_v, [ev], pos, mask=last)
            out_v[pl.ds(i * 16, 16)] = jnp.where(
                pos <= capacity, gv, jnp.float32(0.0))
            return carry
        lax.fori_loop(0, sub // 16, sb, 0)
        pltpu.sync_copy(out_v, out_hbm.at[pl.ds(base, sub)])

    return scan


def kernel(x, W):
    B, T, D = x.shape
    S = B * T
    E = W.shape[0]
    capacity = math.ceil(S / E * CAPACITY_FACTOR)

    x2 = x.reshape(S, D)
    wt = W.T

    idx2, gate2, hist, aux = _tc_call(x2, wt)
    e_flat = idx2.reshape(S)
    g_flat = gate2.reshape(S)
    h_flat = hist.reshape(-1)
    kept = _make_sc_scan(S, capacity)(e_flat, g_flat, h_flat)
    return (e_flat, kept, aux[0])


# speedup vs baseline: 2.1551x; 1.0032x over previous
import functools
import math

import jax
import jax.numpy as jnp
from jax import lax
from jax.experimental import pallas as pl
from jax.experimental.pallas import tpu as pltpu
from jax.experimental.pallas import tpu_sc as plsc

N_EXPERTS = 64
CAPACITY_FACTOR = 1.25
AUX_COEF = 0.01

_CH = 1024
_SUB = 512
_NW = 32


def _tc_body(S, x_ref, wt_ref, idx_ref, gate_ref, hist_ref, aux_ref,
             cnt_acc, p_acc, z_acc):
    i = pl.program_id(0)
    E = wt_ref.shape[1]

    @pl.when(i == 0)
    def _init():
        cnt_acc[...] = jnp.zeros_like(cnt_acc)
        p_acc[...] = jnp.zeros_like(p_acc)
        z_acc[0] = jnp.float32(0.0)

    l = jnp.dot(x_ref[...], wt_ref[...], preferred_element_type=jnp.float32)
    m = jnp.max(l, axis=1, keepdims=True)
    ex = jnp.exp(l - m)
    s = jnp.sum(ex, axis=1, keepdims=True)
    idx = jnp.argmax(l, axis=1).astype(jnp.int32)
    s1 = s[:, 0]
    r1 = (1.0 / s1.reshape(_CH // 128, 128)).reshape(_CH)
    idx_ref[0, 0, :] = idx
    gate_ref[0, 0, :] = r1

    p_acc[...] += jnp.sum(ex * r1[:, None], axis=0, keepdims=True)
    oh = (lax.broadcasted_iota(jnp.int32, l.shape, 1)
          == idx[:, None]).astype(jnp.float32)
    for j in range(_CH // _SUB):
        h = jnp.sum(oh[j * _SUB:(j + 1) * _SUB, :], axis=0, keepdims=True)
        cnt_acc[...] += h
        hist_ref[0, j, :] = h[0].astype(jnp.int32)

    logz = m[:, 0] + jnp.log(s1)
    z_acc[0] += jnp.sum(logz * logz)

    @pl.when(i == pl.num_programs(0) - 1)
    def _fin():
        zl = AUX_COEF * z_acc[0] / S
        lb = (AUX_COEF * E * jnp.sum(cnt_acc[...] * p_acc[...])
              / (jnp.float32(S) * jnp.float32(S)))
        aux_ref[0] = zl + lb


def _tc_call(x2, wt):
    S, D = x2.shape
    E = wt.shape[1]
    nch = S // _CH
    return pl.pallas_call(
        functools.partial(_tc_body, S),
        grid=(nch,),
        in_specs=[
            pl.BlockSpec((_CH, D), lambda i: (i, 0)),
            pl.BlockSpec((D, E), lambda i: (0, 0)),
        ],
        out_specs=[
            pl.BlockSpec((1, 1, _CH), lambda i: (i, 0, 0)),
            pl.BlockSpec((1, 1, _CH), lambda i: (i, 0, 0)),
            pl.BlockSpec((1, _CH // _SUB, E), lambda i: (i, 0, 0)),
            pl.BlockSpec(memory_space=pltpu.MemorySpace.SMEM),
        ],
        out_shape=[
            jax.ShapeDtypeStruct((nch, 1, _CH), jnp.int32),
            jax.ShapeDtypeStruct((nch, 1, _CH), jnp.float32),
            jax.ShapeDtypeStruct((nch, _CH // _SUB, E), jnp.int32),
            jax.ShapeDtypeStruct((1,), jnp.float32),
        ],
        scratch_shapes=[
            pltpu.VMEM((1, E), jnp.float32),
            pltpu.VMEM((1, E), jnp.float32),
            pltpu.SMEM((1,), jnp.float32),
        ],
        compiler_params=pltpu.CompilerParams(
            dimension_semantics=("arbitrary",)),
    )(x2, wt)


def _make_sc_scan(S, capacity):
    E = N_EXPERTS
    sub = S // _NW
    rpt = sub // _SUB
    nhist = S // _SUB
    mesh = plsc.VectorSubcoreMesh(core_axis_name="c", subcore_axis_name="s")

    @functools.partial(
        pl.kernel,
        mesh=mesh,
        compiler_params=pltpu.CompilerParams(needs_layout_passes=False),
        out_type=jax.ShapeDtypeStruct((S,), jnp.float32),
        scratch_types=[
            pltpu.VMEM((nhist * E,), jnp.int32),
            pltpu.VMEM((sub,), jnp.int32),
            pltpu.VMEM((sub,), jnp.float32),
            pltpu.VMEM((E,), jnp.int32),
            pltpu.VMEM((sub,), jnp.float32),
        ],
    )
    def scan(e_hbm, g_hbm, h_hbm, out_hbm, all_v, e_v, g_v, cnt_v, out_v):
        wid = lax.axis_index("s") * 2 + lax.axis_index("c")
        base = wid * sub
        pltpu.sync_copy(h_hbm, all_v)
        pltpu.sync_copy(e_hbm.at[pl.ds(base, sub)], e_v)
        pltpu.sync_copy(g_hbm.at[pl.ds(base, sub)], g_v)

        def bb(t, acc):
            return tuple(a + all_v[pl.ds(t * E + 16 * j, 16)]
                         for j, a in enumerate(acc))
        z16 = jnp.zeros((16,), jnp.int32)
        acc = lax.fori_loop(0, wid * rpt, bb, (z16,) * (E // 16))
        for j in range(E // 16):
            cnt_v[pl.ds(16 * j, 16)] = acc[j]

        def sb(i, carry):
            ev = e_v[pl.ds(i * 16, 16)]
            gv = g_v[pl.ds(i * 16, 16)]
            cb = plsc.load_gather(cnt_v, [ev])
            occ, last = plsc.scan_count(ev)
            pos = cb + occ
            plsc.store_scatter(cnt_v, [ev], pos, mask=last)
            out_v[pl.ds(i * 16, 16)] = jnp.where(
                pos <= capacity, gv, jnp.float32(0.0))
            return carry
        lax.fori_loop(0, sub // 16, sb, 0)
        pltpu.sync_copy(out_v, out_hbm.at[pl.ds(base, sub)])

    return scan


def kernel(x, W):
    B, T, D = x.shape
    S = B * T
    E = W.shape[0]
    capacity = math.ceil(S / E * CAPACITY_FACTOR)

    x2 = x.reshape(S, D)
    wt = W.T

    idx2, gate2, hist, aux = _tc_call(x2, wt)
    e_flat = idx2.reshape(S)
    g_flat = gate2.reshape(S)
    h_flat = hist.reshape(-1)
    kept = _make_sc_scan(S, capacity)(e_flat, g_flat, h_flat)
    return (e_flat, kept, aux[0])


# speedup vs baseline: 2.4480x; 1.1359x over previous
import functools
import math

import jax
import jax.numpy as jnp
from jax import lax
from jax.experimental import pallas as pl
from jax.experimental.pallas import tpu as pltpu
from jax.experimental.pallas import tpu_sc as plsc

N_EXPERTS = 64
CAPACITY_FACTOR = 1.25
AUX_COEF = 0.01

_CH = 1024
_SUB = 512
_NW = 32


def _tc_body(S, x_ref, wt_ref, idx_ref, gate_ref, hist_ref, aux_ref,
             cnt_acc, p_acc, z_acc):
    i = pl.program_id(0)
    E = wt_ref.shape[1]

    @pl.when(i == 0)
    def _init():
        cnt_acc[...] = jnp.zeros_like(cnt_acc)
        p_acc[...] = jnp.zeros_like(p_acc)
        z_acc[0] = jnp.float32(0.0)

    l = jnp.dot(x_ref[...], wt_ref[...], preferred_element_type=jnp.float32)
    m = jnp.max(l, axis=1, keepdims=True)
    ex = jnp.exp(l - m)
    s = jnp.sum(ex, axis=1, keepdims=True)
    eqf = (l == m).astype(jnp.float32)
    iota_col = lax.broadcasted_iota(jnp.int32, (E, 1), 0).astype(jnp.float32)
    idx_f = jnp.dot(eqf, iota_col, preferred_element_type=jnp.float32)
    st = jnp.transpose(s, (1, 0))
    mt = jnp.transpose(m, (1, 0))
    it = jnp.transpose(idx_f, (1, 0))
    rt = 1.0 / st
    idx_ref[0, 0, :] = jnp.minimum(it, E - 1.0).astype(jnp.int32)[0]
    gate_ref[0, 0, :] = rt[0]

    p_acc[...] += jnp.dot(rt, ex, preferred_element_type=jnp.float32)
    for j in range(_CH // _SUB):
        h = jnp.sum(eqf[j * _SUB:(j + 1) * _SUB, :], axis=0, keepdims=True)
        cnt_acc[...] += h
        hist_ref[0, j, :] = h[0].astype(jnp.int32)

    logz = mt + jnp.log(st)
    z_acc[0] += jnp.sum(logz * logz)

    @pl.when(i == pl.num_programs(0) - 1)
    def _fin():
        zl = AUX_COEF * z_acc[0] / S
        lb = (AUX_COEF * E * jnp.sum(cnt_acc[...] * p_acc[...])
              / (jnp.float32(S) * jnp.float32(S)))
        aux_ref[0] = zl + lb


def _tc_call(x2, wt):
    S, D = x2.shape
    E = wt.shape[1]
    nch = S // _CH
    return pl.pallas_call(
        functools.partial(_tc_body, S),
        grid=(nch,),
        in_specs=[
            pl.BlockSpec((_CH, D), lambda i: (i, 0)),
            pl.BlockSpec((D, E), lambda i: (0, 0)),
        ],
        out_specs=[
            pl.BlockSpec((1, 1, _CH), lambda i: (i, 0, 0)),
            pl.BlockSpec((1, 1, _CH), lambda i: (i, 0, 0)),
            pl.BlockSpec((1, _CH // _SUB, E), lambda i: (i, 0, 0)),
            pl.BlockSpec(memory_space=pltpu.MemorySpace.SMEM),
        ],
        out_shape=[
            jax.ShapeDtypeStruct((nch, 1, _CH), jnp.int32),
            jax.ShapeDtypeStruct((nch, 1, _CH), jnp.float32),
            jax.ShapeDtypeStruct((nch, _CH // _SUB, E), jnp.int32),
            jax.ShapeDtypeStruct((1,), jnp.float32),
        ],
        scratch_shapes=[
            pltpu.VMEM((1, E), jnp.float32),
            pltpu.VMEM((1, E), jnp.float32),
            pltpu.SMEM((1,), jnp.float32),
        ],
        compiler_params=pltpu.CompilerParams(
            dimension_semantics=("arbitrary",)),
    )(x2, wt)


def _make_sc_scan(S, capacity):
    E = N_EXPERTS
    sub = S // _NW
    rpt = sub // _SUB
    nhist = S // _SUB
    mesh = plsc.VectorSubcoreMesh(core_axis_name="c", subcore_axis_name="s")

    @functools.partial(
        pl.kernel,
        mesh=mesh,
        compiler_params=pltpu.CompilerParams(needs_layout_passes=False),
        out_type=jax.ShapeDtypeStruct((S,), jnp.float32),
        scratch_types=[
            pltpu.VMEM((nhist * E,), jnp.int32),
            pltpu.VMEM((sub,), jnp.int32),
            pltpu.VMEM((sub,), jnp.float32),
            pltpu.VMEM((E,), jnp.int32),
            pltpu.VMEM((sub,), jnp.float32),
        ],
    )
    def scan(e_hbm, g_hbm, h_hbm, out_hbm, all_v, e_v, g_v, cnt_v, out_v):
        wid = lax.axis_index("s") * 2 + lax.axis_index("c")
        base = wid * sub
        pltpu.sync_copy(h_hbm, all_v)
        pltpu.sync_copy(e_hbm.at[pl.ds(base, sub)], e_v)
        pltpu.sync_copy(g_hbm.at[pl.ds(base, sub)], g_v)

        def bb(t, acc):
            return tuple(a + all_v[pl.ds(t * E + 16 * j, 16)]
                         for j, a in enumerate(acc))
        z16 = jnp.zeros((16,), jnp.int32)
        acc = lax.fori_loop(0, wid * rpt, bb, (z16,) * (E // 16))
        for j in range(E // 16):
            cnt_v[pl.ds(16 * j, 16)] = acc[j]

        def sb(i, carry):
            ev = e_v[pl.ds(i * 16, 16)]
            gv = g_v[pl.ds(i * 16, 16)]
            cb = plsc.load_gather(cnt_v, [ev])
            occ, last = plsc.scan_count(ev)
            pos = cb + occ
            plsc.store_scatter(cnt_v, [ev], pos, mask=last)
            out_v[pl.ds(i * 16, 16)] = jnp.where(
                pos <= capacity, gv, jnp.float32(0.0))
            return carry
        lax.fori_loop(0, sub // 16, sb, 0)
        pltpu.sync_copy(out_v, out_hbm.at[pl.ds(base, sub)])

    return scan


def kernel(x, W):
    B, T, D = x.shape
    S = B * T
    E = W.shape[0]
    capacity = math.ceil(S / E * CAPACITY_FACTOR)

    x2 = x.reshape(S, D)
    wt = W.T

    idx2, gate2, hist, aux = _tc_call(x2, wt)
    e_flat = idx2.reshape(S)
    g_flat = gate2.reshape(S)
    h_flat = hist.reshape(-1)
    kept = _make_sc_scan(S, capacity)(e_flat, g_flat, h_flat)
    return (e_flat, kept, aux[0])
